# Initial kernel scaffold; baseline (speedup 1.0000x reference)
import jax
import jax.numpy as jnp
from jax.experimental import pallas as pl


def kernel(x, edge_index, W1, b1, W2, b2):
    raise NotImplementedError("write your pallas kernel here")


# speedup vs baseline: 30.6981x; 30.6981x over previous
import functools

import jax
import jax.numpy as jnp
from jax import lax
from jax.experimental import pallas as pl
from jax.experimental.pallas import tpu as pltpu
from jax.experimental.pallas import tpu_sc as plsc

NC = 2
NS = 16
NW = NC * NS
LANES = 16

_MESH = dict(core_axis_name="c", subcore_axis_name="s", num_cores=NC,
             num_subcores=NS)


def _ceil_to(x, m):
  return (x + m - 1) // m * m


def _make_deg_kernel(E, NPAD, KD):
  ET = E // NW
  n_chunks = ET // KD
  TPN = NPAD // NS

  mesh = plsc.VectorSubcoreMesh(**_MESH)

  @functools.partial(
      pl.kernel, mesh=mesh,
      out_type=jax.ShapeDtypeStruct((NC * NPAD,), jnp.float32),
      scratch_types=[
          pltpu.VMEM((KD,), jnp.int32),
          pltpu.VMEM((KD,), jnp.float32),
          pltpu.VMEM_SHARED((NPAD,), jnp.float32),
      ],
      name="sc_gcn_degree",
  )
  def deg_kernel(dst_hbm, zeros_hbm, out_hbm, dstb, onesb, acc):
    c = lax.axis_index("c")
    s = lax.axis_index("s")
    def fill(j, _):
      onesb[pl.ds(j * LANES, LANES)] = jnp.full((LANES,), 1.0, jnp.float32)
      return _
    lax.fori_loop(0, KD // LANES, fill, None)
    pltpu.sync_copy(zeros_hbm.at[pl.ds(s * TPN, TPN)],
                    acc.at[pl.ds(s * TPN, TPN)])
    plsc.subcore_barrier()
    wid = c * NS + s
    def chunk(i, _):
      off = wid * ET + i * KD
      pltpu.sync_copy(dst_hbm.at[pl.ds(off, KD)], dstb)
      pltpu.sync_copy(onesb, acc.at[dstb], add=True)
      return _
    lax.fori_loop(0, n_chunks, chunk, None)
    plsc.subcore_barrier()
    pltpu.sync_copy(acc.at[pl.ds(s * TPN, TPN)],
                    out_hbm.at[pl.ds(c * NPAD + s * TPN, TPN)])

  return deg_kernel


def _tc_prep_body(degp0_ref, degp1_ref, x_ref, w1_ref, hs_ref, dinv_ref):
  deg = degp0_ref[...] + degp1_ref[...] + 1.0
  dinv = lax.rsqrt(deg)
  h = jnp.dot(x_ref[...], w1_ref[...], preferred_element_type=jnp.float32)
  hs_ref[...] = h * dinv
  dinv_ref[...] = dinv


def _tc_prep(degp0, degp1, x, W1, N, C):
  BR = 1000
  grid = N // BR
  return pl.pallas_call(
      _tc_prep_body,
      grid=(grid,),
      in_specs=[
          pl.BlockSpec((BR, 1), lambda i: (i, 0)),
          pl.BlockSpec((BR, 1), lambda i: (i, 0)),
          pl.BlockSpec((BR, C), lambda i: (i, 0)),
          pl.BlockSpec((C, C), lambda i: (0, 0)),
      ],
      out_specs=[
          pl.BlockSpec((BR, C), lambda i: (i, 0)),
          pl.BlockSpec((BR, 1), lambda i: (i, 0)),
      ],
      out_shape=[
          jax.ShapeDtypeStruct((N, C), jnp.float32),
          jax.ShapeDtypeStruct((N, 1), jnp.float32),
      ],
      name="tc_gcn_prep",
  )(degp0, degp1, x, W1)


def _make_agg_kernel(E, NPAD, C, K):
  ET = E // NW
  n_chunks = ET // K
  TPN = NPAD // NS

  mesh = plsc.VectorSubcoreMesh(**_MESH)

  @functools.partial(
      pl.kernel, mesh=mesh,
      out_type=jax.ShapeDtypeStruct((NC * NPAD, C), jnp.float32),
      scratch_types=[
          pltpu.VMEM((K,), jnp.int32),
          pltpu.VMEM((K,), jnp.int32),
          pltpu.VMEM((K, C), jnp.float32),
          pltpu.VMEM_SHARED((NPAD, C), jnp.float32),
          pltpu.SemaphoreType.DMA,
      ],
      name="sc_gcn_edge_agg",
  )
  def agg_kernel(src_hbm, dst_hbm, hs_hbm, zeros_hbm, out_hbm,
                 srcb, dstb, rows, acc, sem):
    c = lax.axis_index("c")
    s = lax.axis_index("s")
    pltpu.sync_copy(zeros_hbm.at[pl.ds(s * TPN, TPN)],
                    acc.at[pl.ds(s * TPN, TPN)])
    plsc.subcore_barrier()
    wid = c * NS + s
    def chunk(i, _):
      off = wid * ET + i * K
      pltpu.sync_copy(src_hbm.at[pl.ds(off, K)], srcb)
      pltpu.sync_copy(dst_hbm.at[pl.ds(off, K)], dstb)
      pltpu.async_copy(hs_hbm.at[srcb], rows, sem).wait()
      pltpu.sync_copy(rows, acc.at[dstb], add=True)
      return _
    lax.fori_loop(0, n_chunks, chunk, None)
    plsc.subcore_barrier()
    pltpu.sync_copy(acc.at[pl.ds(s * TPN, TPN)],
                    out_hbm.at[pl.ds(c * NPAD + s * TPN, TPN)])

  return agg_kernel


def _tc_mid_body(p0_ref, p1_ref, hs_ref, dinv_ref, b1_ref, w2_ref, ys_ref):
  dinv = dinv_ref[...]
  z = dinv * (p0_ref[...] + p1_ref[...] + hs_ref[...]) + b1_ref[...]
  z = jnp.maximum(z, 0.0)
  y = jnp.dot(z, w2_ref[...], preferred_element_type=jnp.float32)
  ys_ref[...] = y * dinv


def _tc_mid(p0, p1, hs, dinv, b1, W2, N, C):
  BR = 1000
  grid = N // BR
  return pl.pallas_call(
      _tc_mid_body,
      grid=(grid,),
      in_specs=[
          pl.BlockSpec((BR, C), lambda i: (i, 0)),
          pl.BlockSpec((BR, C), lambda i: (i, 0)),
          pl.BlockSpec((BR, C), lambda i: (i, 0)),
          pl.BlockSpec((BR, 1), lambda i: (i, 0)),
          pl.BlockSpec((1, C), lambda i: (0, 0)),
          pl.BlockSpec((C, 1), lambda i: (0, 0)),
      ],
      out_specs=pl.BlockSpec((BR, 1), lambda i: (i, 0)),
      out_shape=jax.ShapeDtypeStruct((N, 1), jnp.float32),
      name="tc_gcn_mid",
  )(p0, p1, hs, dinv, b1, W2)


def _make_agg2_kernel(E, NPAD, K2):
  ET = E // NS
  n_chunks = ET // K2
  TPN = NPAD // NS

  mesh = plsc.VectorSubcoreMesh(**_MESH)

  @functools.partial(
      pl.kernel, mesh=mesh,
      out_type=jax.ShapeDtypeStruct((NPAD,), jnp.float32),
      scratch_types=[
          pltpu.VMEM((K2,), jnp.int32),
          pltpu.VMEM((K2,), jnp.int32),
          pltpu.VMEM((K2,), jnp.float32),
          pltpu.VMEM((TPN,), jnp.float32),
          pltpu.VMEM((TPN,), jnp.float32),
          pltpu.VMEM((TPN,), jnp.float32),
          pltpu.VMEM((16,), jnp.float32),
          pltpu.VMEM_SHARED((NPAD,), jnp.float32),
          pltpu.SemaphoreType.DMA,
      ],
      name="sc_gcn_scalar_agg",
  )
  def agg2_kernel(src_hbm, dst_hbm, ys_hbm, dinv_hbm, b2_hbm, zeros_hbm,
                  out_hbm, srcb, dstb, vals, ybuf, dbuf, obuf, b2b, acc, sem):
    c = lax.axis_index("c")
    s = lax.axis_index("s")

    @pl.when(c == 0)
    def _core0():
      pltpu.sync_copy(zeros_hbm.at[pl.ds(s * TPN, TPN)],
                      acc.at[pl.ds(s * TPN, TPN)])
      plsc.subcore_barrier()

      def chunk(i, _):
        off = s * ET + i * K2
        pltpu.sync_copy(src_hbm.at[pl.ds(off, K2)], srcb)
        pltpu.sync_copy(dst_hbm.at[pl.ds(off, K2)], dstb)
        pltpu.async_copy(ys_hbm.at[srcb], vals, sem).wait()
        pltpu.sync_copy(vals, acc.at[dstb], add=True)
        return _
      lax.fori_loop(0, n_chunks, chunk, None)
      plsc.subcore_barrier()

      pltpu.sync_copy(acc.at[pl.ds(s * TPN, TPN)], obuf)
      pltpu.sync_copy(ys_hbm.at[pl.ds(s * TPN, TPN)], ybuf)
      pltpu.sync_copy(dinv_hbm.at[pl.ds(s * TPN, TPN)], dbuf)
      pltpu.sync_copy(b2_hbm, b2b)
      def fin(j, _):
        sl = pl.ds(j * LANES, LANES)
        a = obuf[sl] + ybuf[sl]
        obuf[sl] = a * dbuf[sl] + b2b[...]
        return _
      lax.fori_loop(0, TPN // LANES, fin, None)
      pltpu.sync_copy(obuf, out_hbm.at[pl.ds(s * TPN, TPN)])

  return agg2_kernel


def kernel(x, edge_index, W1, b1, W2, b2):
  N, C = x.shape
  E = edge_index.shape[1]
  NPAD = _ceil_to(N, NS * LANES)

  KD = 2000
  K = 200
  K2 = 2000

  EPAD = _ceil_to(E, NW * KD)
  EPAD = _ceil_to(EPAD, NW * K)
  EPAD = _ceil_to(EPAD, NS * K2)

  ei = edge_index.astype(jnp.int32)
  src = ei[0]
  dst = ei[1]
  if EPAD != E:
    src = jnp.pad(src, (0, EPAD - E))
    dst = jnp.pad(dst, (0, EPAD - E), constant_values=NPAD - 1)

  zeros_row = jnp.zeros((NPAD,), jnp.float32)
  zeros_mat = jnp.zeros((NPAD, C), jnp.float32)

  degp = _make_deg_kernel(EPAD, NPAD, KD)(dst, zeros_row)
  degp = degp.reshape(NC, NPAD)
  degp0 = degp[0, :N].reshape(N, 1)
  degp1 = degp[1, :N].reshape(N, 1)

  hs, dinv = _tc_prep(degp0, degp1, x, W1, N, C)

  parts = _make_agg_kernel(EPAD, NPAD, C, K)(src, dst, hs, zeros_mat)
  parts = parts.reshape(NC, NPAD, C)
  p0 = parts[0, :N, :]
  p1 = parts[1, :N, :]

  ys = _tc_mid(p0, p1, hs, dinv, b1.reshape(1, C), W2, N, C)

  ys_pad = jnp.pad(ys[:, 0], (0, NPAD - N))
  dinv_pad = jnp.pad(dinv[:, 0], (0, NPAD - N))
  b2_b = jnp.broadcast_to(b2.astype(jnp.float32), (16,))

  out = _make_agg2_kernel(EPAD, NPAD, K2)(src, dst, ys_pad, dinv_pad, b2_b,
                                          zeros_row)
  return out[:N]
